# baseline (speedup 1.0000x reference)
import functools

import jax
import jax.numpy as jnp
from jax import lax
from jax.experimental import pallas as pl
from jax.experimental.pallas import tpu as pltpu
from jax.experimental.pallas import tpu_sc as plsc

NUM_EMB = 1_000_000
DIM = 64
BATCH = 4096
HIST = 200
NC, NS = 2, 16
NW = NC * NS
B_PER_W = BATCH // NW
T_TILES = HIST // 8
NSLOT = 4


def _emb_body(wt_hbm, weight_hbm, out_hbm, stage_v, raw_v, tr_v, gsem, wsem):
    wid = lax.axis_index("s") * NC + lax.axis_index("c")
    b0 = wid * B_PER_W

    for ti in range(T_TILES):
        pltpu.sync_copy(
            wt_hbm.at[pl.ds(ti * (NW * 1024) + wid * 1024, 1024)],
            stage_v.at[pl.ds(ti * 1024, 1024)],
        )

    def gather_copy(t, s):
        return pltpu.make_async_copy(
            weight_hbm.at[stage_v.at[pl.ds(t * B_PER_W, B_PER_W)]],
            raw_v.at[s],
            gsem.at[s],
        )

    def write_copy(t, s):
        return pltpu.make_async_copy(
            tr_v.at[s],
            out_hbm.at[t, :, pl.ds(b0, B_PER_W)],
            wsem.at[s],
        )

    iotas = [lax.iota(jnp.int32, 16) + k for k in range(0, B_PER_W, 16)]

    def transpose(s):
        def tbody(c, carry):
            colv = jnp.full((16,), 0, jnp.int32) + c
            for k in range(B_PER_W // 16):
                v = plsc.load_gather(raw_v.at[s], [iotas[k], colv])
                tr_v[s, c, pl.ds(k * 16, 16)] = v
            return carry

        lax.fori_loop(0, DIM, tbody, 0)

    def step(t, s, wait_write, refill):
        gather_copy(t, s).wait()
        if wait_write:
            write_copy(t, s).wait()
        transpose(s)
        write_copy(t, s).start()
        if refill:
            gather_copy(t + NSLOT, s).start()

    for s in range(NSLOT):
        gather_copy(s, s).start()
    for s in range(NSLOT):
        step(s, s, False, True)

    def body(i, carry):
        for s in range(NSLOT):
            step(NSLOT * i + s, s, True, True)
        return carry

    lax.fori_loop(1, HIST // NSLOT - 1, body, 0)

    for s in range(NSLOT):
        step(HIST - NSLOT + s, s, True, False)
    for s in range(NSLOT):
        write_copy(HIST - NSLOT + s, s).wait()


@functools.partial(jax.jit)
def _embedding_gather(words_flat, weight):
    mesh = plsc.VectorSubcoreMesh(core_axis_name="c", subcore_axis_name="s")
    f = pl.kernel(
        _emb_body,
        out_type=jax.ShapeDtypeStruct((HIST, DIM, BATCH), jnp.float32),
        mesh=mesh,
        scratch_types=[
            pltpu.VMEM((HIST * B_PER_W,), jnp.int32),
            pltpu.VMEM((NSLOT, B_PER_W, DIM), jnp.float32),
            pltpu.VMEM((NSLOT, DIM, B_PER_W), jnp.float32),
            pltpu.SemaphoreType.DMA((NSLOT,)),
            pltpu.SemaphoreType.DMA((NSLOT,)),
        ],
        compiler_params=pltpu.CompilerParams(
            use_tc_tiling_on_sc=False, needs_layout_passes=False
        ),
    )
    return f(words_flat, weight)


def kernel(words, weight):
    wt = words.T.reshape(T_TILES, 8, NW, B_PER_W)
    wt = wt.transpose(0, 2, 1, 3).reshape(-1).astype(jnp.int32)
    out_phys = _embedding_gather(wt, weight)
    return out_phys.transpose(2, 0, 1)


# speedup vs baseline: 1.1517x; 1.1517x over previous
import functools

import jax
import jax.numpy as jnp
from jax import lax
from jax.experimental import pallas as pl
from jax.experimental.pallas import tpu as pltpu
from jax.experimental.pallas import tpu_sc as plsc
from jax._src.pjit import with_layout_constraint
from jax._src.layout import Layout

NUM_EMB = 1_000_000
DIM = 64
BATCH = 4096
HIST = 200
NC, NS = 2, 16
NW = NC * NS
B_PER_W = BATCH // NW
HIST_PAD = 208
CHUNK = 40
K = HIST // CHUNK
T_TILES = HIST // 8
COLB = 512


def _tr_body(x_ref, o_ref):
    o_ref[:, :DIM] = x_ref[...].T


def _weight_rowmajor(wT):
    return pl.pallas_call(
        _tr_body,
        grid=(pl.cdiv(NUM_EMB, COLB),),
        in_specs=[pl.BlockSpec((DIM, COLB), lambda i: (0, i))],
        out_specs=pl.BlockSpec((COLB, 128), lambda i: (i, 0)),
        out_shape=jax.ShapeDtypeStruct((NUM_EMB, 128), jnp.float32),
    )(wT)


def _emb_body(words_hbm, table_hbm, out_hbm, stage_v, idx_v, rows_v, gsem, wsem):
    wid = lax.axis_index("s") * NC + lax.axis_index("c")
    b0 = wid * B_PER_W

    for ti in range(T_TILES):
        pltpu.sync_copy(
            words_hbm.at[pl.ds(ti * (NW * 1024) + wid * 1024, 1024)],
            stage_v.at[pl.ds(ti * 1024, 1024)],
        )

    lanes = lax.iota(jnp.int32, 16) * 128

    def transpose_body(bb, carry):
        for t0 in range(0, HIST_PAD, 16):
            v = plsc.load_gather(stage_v, [lanes + (t0 * 128 + bb)])
            idx_v[pl.ds(bb * HIST_PAD + t0, 16)] = v
        return carry

    lax.fori_loop(0, B_PER_W, transpose_body, 0)

    def gather_copy(g, buf, j):
        return pltpu.make_async_copy(
            table_hbm.at[idx_v.at[pl.ds(g * HIST_PAD + j * CHUNK, CHUNK)]],
            rows_v.at[buf, pl.ds(j * CHUNK, CHUNK)],
            gsem.at[buf],
        )

    def start_group(g, buf):
        for j in range(K):
            gather_copy(g, buf, j).start()

    def wait_group(g, buf):
        for j in range(K):
            gather_copy(g, buf, j).wait()

    def write_copy(g, buf):
        return pltpu.make_async_copy(
            rows_v.at[buf, :, pl.ds(0, DIM)],
            out_hbm.at[b0 + g],
            wsem.at[buf],
        )

    start_group(0, 0)
    start_group(1, 1)

    def body(i, carry):
        g = 2 * i
        for buf in (0, 1):
            wait_group(g + buf, buf)
            write_copy(g + buf, buf).start()
            write_copy(g + buf, buf).wait()
            start_group(g + buf + 2, buf)
        return carry

    lax.fori_loop(0, (B_PER_W - 2) // 2, body, 0)

    for buf in (0, 1):
        g = B_PER_W - 2 + buf
        wait_group(g, buf)
        write_copy(g, buf).start()
    for buf in (0, 1):
        write_copy(B_PER_W - 2 + buf, buf).wait()


@functools.partial(jax.jit)
def _embedding_gather(words_flat, table):
    mesh = plsc.VectorSubcoreMesh(core_axis_name="c", subcore_axis_name="s")
    f = pl.kernel(
        _emb_body,
        out_type=jax.ShapeDtypeStruct((BATCH, HIST, DIM), jnp.float32),
        mesh=mesh,
        scratch_types=[
            pltpu.VMEM((B_PER_W * HIST_PAD,), jnp.int32),
            pltpu.VMEM((B_PER_W * HIST_PAD,), jnp.int32),
            pltpu.VMEM((2, HIST, 128), jnp.float32),
            pltpu.SemaphoreType.DMA((2,)),
            pltpu.SemaphoreType.DMA((2,)),
        ],
        compiler_params=pltpu.CompilerParams(
            use_tc_tiling_on_sc=False, needs_layout_passes=False
        ),
    )
    return f(words_flat, table)


def kernel(words, weight):
    table = _weight_rowmajor(weight.T)
    wt = words.T.reshape(T_TILES, 8, NW, B_PER_W)
    wt = wt.transpose(0, 2, 1, 3).reshape(-1).astype(jnp.int32)
    out = _embedding_gather(wt, table)
    return with_layout_constraint(out, Layout(major_to_minor=(0, 1, 2)))


# speedup vs baseline: 2.4292x; 2.1093x over previous
import functools

import jax
import jax.numpy as jnp
from jax import lax
from jax.experimental import pallas as pl
from jax.experimental.pallas import tpu as pltpu
from jax.experimental.pallas import tpu_sc as plsc
from jax._src.pjit import with_layout_constraint
from jax._src.layout import Layout

NUM_EMB = 1_000_000
DIM = 64
BATCH = 4096
HIST = 200
NC, NS = 2, 16
NW = NC * NS
B_PER_W = BATCH // NW
HIST_PAD = 208
CHUNK = 40
K = HIST // CHUNK
T_TILES = HIST // 8
COLB = 8192


def _tr_body(x_ref, o_ref):
    o_ref[:, :DIM] = x_ref[...].T


def _weight_rowmajor(wT):
    return pl.pallas_call(
        _tr_body,
        grid=(pl.cdiv(NUM_EMB, COLB),),
        in_specs=[pl.BlockSpec((DIM, COLB), lambda i: (0, i))],
        out_specs=pl.BlockSpec((COLB, 128), lambda i: (i, 0)),
        out_shape=jax.ShapeDtypeStruct((NUM_EMB, 128), jnp.float32),
    )(wT)


def _emb_body(words_hbm, table_hbm, out_hbm, stage_v, idx_v, rows_v, gsem, wsem):
    wid = lax.axis_index("s") * NC + lax.axis_index("c")
    b0 = wid * B_PER_W

    for ti in range(T_TILES):
        pltpu.sync_copy(
            words_hbm.at[pl.ds(ti * (NW * 1024) + wid * 1024, 1024)],
            stage_v.at[pl.ds(ti * 1024, 1024)],
        )

    lanes = lax.iota(jnp.int32, 16) * 128

    def transpose_body(bb, carry):
        for t0 in range(0, HIST_PAD, 16):
            v = plsc.load_gather(stage_v, [lanes + (t0 * 128 + bb)])
            idx_v[pl.ds(bb * HIST_PAD + t0, 16)] = v
        return carry

    lax.fori_loop(0, B_PER_W, transpose_body, 0)

    def gather_copy(g, buf, j):
        return pltpu.make_async_copy(
            table_hbm.at[idx_v.at[pl.ds(g * HIST_PAD + j * CHUNK, CHUNK)]],
            rows_v.at[buf, pl.ds(j * CHUNK, CHUNK)],
            gsem.at[buf],
        )

    def start_group(g, buf):
        for j in range(K):
            gather_copy(g, buf, j).start()

    def wait_group(g, buf):
        for j in range(K):
            gather_copy(g, buf, j).wait()

    def write_copy(g, buf):
        return pltpu.make_async_copy(
            rows_v.at[buf, :, pl.ds(0, DIM)],
            out_hbm.at[b0 + g],
            wsem.at[buf],
        )

    start_group(0, 0)
    start_group(1, 1)

    def body(i, carry):
        g = 2 * i
        for buf in (0, 1):
            wait_group(g + buf, buf)
            write_copy(g + buf, buf).start()
            write_copy(g + buf, buf).wait()
            start_group(g + buf + 2, buf)
        return carry

    lax.fori_loop(0, (B_PER_W - 2) // 2, body, 0)

    for buf in (0, 1):
        g = B_PER_W - 2 + buf
        wait_group(g, buf)
        write_copy(g, buf).start()
    for buf in (0, 1):
        write_copy(B_PER_W - 2 + buf, buf).wait()


@functools.partial(jax.jit)
def _embedding_gather(words_flat, table):
    mesh = plsc.VectorSubcoreMesh(core_axis_name="c", subcore_axis_name="s")
    f = pl.kernel(
        _emb_body,
        out_type=jax.ShapeDtypeStruct((BATCH, HIST, DIM), jnp.float32),
        mesh=mesh,
        scratch_types=[
            pltpu.VMEM((B_PER_W * HIST_PAD,), jnp.int32),
            pltpu.VMEM((B_PER_W * HIST_PAD,), jnp.int32),
            pltpu.VMEM((2, HIST, 128), jnp.float32),
            pltpu.SemaphoreType.DMA((2,)),
            pltpu.SemaphoreType.DMA((2,)),
        ],
        compiler_params=pltpu.CompilerParams(
            use_tc_tiling_on_sc=False, needs_layout_passes=False
        ),
    )
    return f(words_flat, table)


def kernel(words, weight):
    table = _weight_rowmajor(weight.T)
    wt = words.T.reshape(T_TILES, 8, NW, B_PER_W)
    wt = wt.transpose(0, 2, 1, 3).reshape(-1).astype(jnp.int32)
    out = _embedding_gather(wt, table)
    return with_layout_constraint(out, Layout(major_to_minor=(0, 1, 2)))


# speedup vs baseline: 2.4880x; 1.0242x over previous
import functools

import jax
import jax.numpy as jnp
from jax import lax
from jax.experimental import pallas as pl
from jax.experimental.pallas import tpu as pltpu
from jax.experimental.pallas import tpu_sc as plsc
from jax._src.pjit import with_layout_constraint
from jax._src.layout import Layout

NUM_EMB = 1_000_000
DIM = 64
BATCH = 4096
HIST = 200
NC, NS = 2, 16
NW = NC * NS
B_PER_W = BATCH // NW
HIST_PAD = 208
CHUNK = 40
K = HIST // CHUNK
T_TILES = HIST // 8
COLB = 16384


def _tr_body(x_ref, o_ref):
    o_ref[:, :DIM] = x_ref[...].T


def _weight_rowmajor(wT):
    return pl.pallas_call(
        _tr_body,
        grid=(pl.cdiv(NUM_EMB, COLB),),
        in_specs=[pl.BlockSpec((DIM, COLB), lambda i: (0, i))],
        out_specs=pl.BlockSpec((COLB, 128), lambda i: (i, 0)),
        out_shape=jax.ShapeDtypeStruct((NUM_EMB, 128), jnp.float32),
    )(wT)


def _emb_body(words_hbm, table_hbm, out_hbm, stage_v, idx_v, rows_v, gsem, wsem):
    wid = lax.axis_index("s") * NC + lax.axis_index("c")
    b0 = wid * B_PER_W

    for ti in range(T_TILES):
        pltpu.sync_copy(
            words_hbm.at[pl.ds(ti * (NW * 1024) + wid * 1024, 1024)],
            stage_v.at[pl.ds(ti * 1024, 1024)],
        )

    lanes = lax.iota(jnp.int32, 16) * 128

    def transpose_body(bb, carry):
        for t0 in range(0, HIST_PAD, 16):
            v = plsc.load_gather(stage_v, [lanes + (t0 * 128 + bb)])
            idx_v[pl.ds(bb * HIST_PAD + t0, 16)] = v
        return carry

    lax.fori_loop(0, B_PER_W, transpose_body, 0)

    def gather_copy(g, buf, j):
        return pltpu.make_async_copy(
            table_hbm.at[idx_v.at[pl.ds(g * HIST_PAD + j * CHUNK, CHUNK)]],
            rows_v.at[buf, pl.ds(j * CHUNK, CHUNK)],
            gsem.at[buf],
        )

    def start_group(g, buf):
        for j in range(K):
            gather_copy(g, buf, j).start()

    def wait_group(g, buf):
        for j in range(K):
            gather_copy(g, buf, j).wait()

    def write_copy(g, buf):
        return pltpu.make_async_copy(
            rows_v.at[buf, :, pl.ds(0, DIM)],
            out_hbm.at[b0 + g],
            wsem.at[buf],
        )

    start_group(0, 0)
    start_group(1, 1)

    def body(i, carry):
        g = 2 * i
        for buf in (0, 1):
            wait_group(g + buf, buf)
            write_copy(g + buf, buf).start()
            write_copy(g + buf, buf).wait()
            start_group(g + buf + 2, buf)
        return carry

    lax.fori_loop(0, (B_PER_W - 2) // 2, body, 0)

    for buf in (0, 1):
        g = B_PER_W - 2 + buf
        wait_group(g, buf)
        write_copy(g, buf).start()
    for buf in (0, 1):
        write_copy(B_PER_W - 2 + buf, buf).wait()


@functools.partial(jax.jit)
def _embedding_gather(words_flat, table):
    mesh = plsc.VectorSubcoreMesh(core_axis_name="c", subcore_axis_name="s")
    f = pl.kernel(
        _emb_body,
        out_type=jax.ShapeDtypeStruct((BATCH, HIST, DIM), jnp.float32),
        mesh=mesh,
        scratch_types=[
            pltpu.VMEM((B_PER_W * HIST_PAD,), jnp.int32),
            pltpu.VMEM((B_PER_W * HIST_PAD,), jnp.int32),
            pltpu.VMEM((2, HIST, 128), jnp.float32),
            pltpu.SemaphoreType.DMA((2,)),
            pltpu.SemaphoreType.DMA((2,)),
        ],
        compiler_params=pltpu.CompilerParams(
            use_tc_tiling_on_sc=False, needs_layout_passes=False
        ),
    )
    return f(words_flat, table)


def kernel(words, weight):
    table = _weight_rowmajor(weight.T)
    wt = words.T.reshape(T_TILES, 8, NW, B_PER_W)
    wt = wt.transpose(0, 2, 1, 3).reshape(-1).astype(jnp.int32)
    out = _embedding_gather(wt, table)
    return with_layout_constraint(out, Layout(major_to_minor=(0, 1, 2)))
